# baseline (device time: 154119 ns/iter reference)
import jax
import jax.numpy as jnp
from jax import lax
from jax.experimental import pallas as pl
from jax.experimental.pallas import tpu as pltpu

N_Y = 4
M_PER = 1024
D = 1024
EPS = 1e-6


def kernel(partial, gamma):
    part = partial.reshape(N_Y * M_PER, D)
    gamma2 = gamma.reshape(1, D)

    def body(part_ref, gamma_ref, out_ref, comm_ref, send_sems, recv_sems):
        my_x = lax.axis_index("x")
        my_y = lax.axis_index("y")
        my_z = lax.axis_index("z")
        left = (my_y - 1) % N_Y
        right = (my_y + 1) % N_Y

        barrier_sem = pltpu.get_barrier_semaphore()
        for nbr in (left, right):
            pl.semaphore_signal(
                barrier_sem, inc=1,
                device_id=(my_x, nbr, my_z),
                device_id_type=pl.DeviceIdType.MESH,
            )
        pl.semaphore_wait(barrier_sem, 2)

        first = (my_y - 1) % N_Y
        comm_ref[0] = part_ref[pl.ds(first * M_PER, M_PER), :]

        for s in range(N_Y - 1):
            send_slot = s % 2
            recv_slot = (s + 1) % 2
            rdma = pltpu.make_async_remote_copy(
                src_ref=comm_ref.at[send_slot],
                dst_ref=comm_ref.at[recv_slot],
                send_sem=send_sems.at[send_slot],
                recv_sem=recv_sems.at[recv_slot],
                device_id=(my_x, right, my_z),
                device_id_type=pl.DeviceIdType.MESH,
            )
            rdma.start()
            rdma.wait()

            chunk = (my_y - 2 - s) % N_Y
            contrib = part_ref[pl.ds(chunk * M_PER, M_PER), :]
            if s < N_Y - 2:
                comm_ref[recv_slot] = comm_ref[recv_slot] + contrib
            else:
                y = comm_ref[recv_slot] + contrib
                ms = jnp.mean(y * y, axis=-1, keepdims=True)
                out_ref[...] = y * lax.rsqrt(ms + EPS) * gamma_ref[...]

    return pl.pallas_call(
        body,
        out_shape=jax.ShapeDtypeStruct((M_PER, D), jnp.float32),
        in_specs=[
            pl.BlockSpec(memory_space=pltpu.VMEM),
            pl.BlockSpec(memory_space=pltpu.VMEM),
        ],
        out_specs=pl.BlockSpec(memory_space=pltpu.VMEM),
        scratch_shapes=[
            pltpu.VMEM((2, M_PER, D), jnp.float32),
            pltpu.SemaphoreType.DMA((2,)),
            pltpu.SemaphoreType.DMA((2,)),
        ],
        compiler_params=pltpu.CompilerParams(collective_id=0),
    )(part, gamma2)


# device time: 153283 ns/iter; 1.0055x vs baseline; 1.0055x over previous
import jax
import jax.numpy as jnp
from jax import lax
from jax.experimental import pallas as pl
from jax.experimental.pallas import tpu as pltpu

N_Y = 4
M_PER = 1024
H = M_PER // 2
D = 1024
EPS = 1e-6


def kernel(partial, gamma):
    part = partial.reshape(N_Y * M_PER, D)
    gamma2 = gamma.reshape(1, D)

    def body(part_ref, gamma_ref, out_ref,
             comm_r, comm_l, send_r, recv_r, send_l, recv_l):
        my_x = lax.axis_index("x")
        my_y = lax.axis_index("y")
        my_z = lax.axis_index("z")
        left = (my_y - 1) % N_Y
        right = (my_y + 1) % N_Y

        barrier_sem = pltpu.get_barrier_semaphore()
        for nbr in (left, right):
            pl.semaphore_signal(
                barrier_sem, inc=1,
                device_id=(my_x, nbr, my_z),
                device_id_type=pl.DeviceIdType.MESH,
            )
        pl.semaphore_wait(barrier_sem, 2)

        seed_r = (my_y - 1) % N_Y
        seed_l = (my_y + 1) % N_Y
        comm_r[0] = part_ref[pl.ds(seed_r * M_PER, H), :]
        comm_l[0] = part_ref[pl.ds(seed_l * M_PER + H, H), :]

        for s in range(N_Y - 1):
            snd = s % 2
            rcv = (s + 1) % 2
            rdma_r = pltpu.make_async_remote_copy(
                src_ref=comm_r.at[snd],
                dst_ref=comm_r.at[rcv],
                send_sem=send_r.at[snd],
                recv_sem=recv_r.at[rcv],
                device_id=(my_x, right, my_z),
                device_id_type=pl.DeviceIdType.MESH,
            )
            rdma_l = pltpu.make_async_remote_copy(
                src_ref=comm_l.at[snd],
                dst_ref=comm_l.at[rcv],
                send_sem=send_l.at[snd],
                recv_sem=recv_l.at[rcv],
                device_id=(my_x, left, my_z),
                device_id_type=pl.DeviceIdType.MESH,
            )
            rdma_r.start()
            rdma_l.start()
            rdma_r.wait()
            rdma_l.wait()

            ch_r = (my_y - 2 - s) % N_Y
            ch_l = (my_y + 2 + s) % N_Y
            top = part_ref[pl.ds(ch_r * M_PER, H), :]
            bot = part_ref[pl.ds(ch_l * M_PER + H, H), :]
            if s < N_Y - 2:
                comm_r[rcv] = comm_r[rcv] + top
                comm_l[rcv] = comm_l[rcv] + bot
            else:
                y_t = comm_r[rcv] + top
                y_b = comm_l[rcv] + bot
                ms_t = jnp.mean(y_t * y_t, axis=-1, keepdims=True)
                ms_b = jnp.mean(y_b * y_b, axis=-1, keepdims=True)
                out_ref[pl.ds(0, H), :] = (
                    y_t * lax.rsqrt(ms_t + EPS) * gamma_ref[...]
                )
                out_ref[pl.ds(H, H), :] = (
                    y_b * lax.rsqrt(ms_b + EPS) * gamma_ref[...]
                )

    return pl.pallas_call(
        body,
        out_shape=jax.ShapeDtypeStruct((M_PER, D), jnp.float32),
        in_specs=[
            pl.BlockSpec(memory_space=pltpu.VMEM),
            pl.BlockSpec(memory_space=pltpu.VMEM),
        ],
        out_specs=pl.BlockSpec(memory_space=pltpu.VMEM),
        scratch_shapes=[
            pltpu.VMEM((2, H, D), jnp.float32),
            pltpu.VMEM((2, H, D), jnp.float32),
            pltpu.SemaphoreType.DMA((2,)),
            pltpu.SemaphoreType.DMA((2,)),
            pltpu.SemaphoreType.DMA((2,)),
            pltpu.SemaphoreType.DMA((2,)),
        ],
        compiler_params=pltpu.CompilerParams(collective_id=0),
    )(part, gamma2)


# device time: 64908 ns/iter; 2.3744x vs baseline; 2.3615x over previous
import jax
import jax.numpy as jnp
from jax import lax
from jax.experimental import pallas as pl
from jax.experimental.pallas import tpu as pltpu

N_Y = 4
N_P = 8
M_PER = 1024
R = M_PER // N_P
D = 1024
EPS = 1e-6


def kernel(partial, gamma):
    part = partial.reshape(N_Y * M_PER, D)
    gamma2 = gamma.reshape(1, D)

    def body(part_ref, gamma_ref, out_ref,
             ybuf, ysend, yrecv, rsend, rrecv, lsend, lrecv):
        my_x = lax.axis_index("x")
        my_y = lax.axis_index("y")
        my_z = lax.axis_index("z")

        p = jnp.where(my_x == 0, my_z, 7 - my_z)

        def coords_of(pos):
            xx = pos // 4
            zz = jnp.where(xx == 0, pos % 4, 7 - pos)
            return xx, zz

        nxt_x, nxt_z = coords_of((p + 1) % N_P)
        prv_x, prv_z = coords_of((p - 1) % N_P)

        barrier_sem = pltpu.get_barrier_semaphore()
        for o in range(1, N_Y):
            pl.semaphore_signal(
                barrier_sem, inc=1,
                device_id=(my_x, (my_y + o) % N_Y, my_z),
                device_id_type=pl.DeviceIdType.MESH,
            )
        for dev in ((nxt_x, my_y, nxt_z), (prv_x, my_y, prv_z)):
            pl.semaphore_signal(
                barrier_sem, inc=1,
                device_id=dev, device_id_type=pl.DeviceIdType.MESH,
            )
        pl.semaphore_wait(barrier_sem, 5)

        y_sends = []
        for o in range(1, N_Y):
            r = (my_y + o) % N_Y
            rd = pltpu.make_async_remote_copy(
                src_ref=part_ref.at[pl.ds(r * M_PER + p * R, R), :],
                dst_ref=ybuf.at[o - 1],
                send_sem=ysend.at[o - 1],
                recv_sem=yrecv.at[o - 1],
                device_id=(my_x, r, my_z),
                device_id_type=pl.DeviceIdType.MESH,
            )
            rd.start()
            y_sends.append(rd)
        for o in range(1, N_Y):
            y_sends[o - 1].wait_recv()

        own = part_ref[pl.ds(my_y * M_PER + p * R, R), :]
        y_loc = own + ybuf[0] + ybuf[1] + ybuf[2]
        ms = jnp.mean(y_loc * y_loc, axis=-1, keepdims=True)
        out_ref[pl.ds(p * R, R), :] = (
            y_loc * lax.rsqrt(ms + EPS) * gamma_ref[...]
        )

        ring_sends = []
        for s in range(4):
            if s < 3:
                q_out_r = (p - s) % N_P
                q_in_r = (p - 1 - s) % N_P
                send_r = pltpu.make_async_remote_copy(
                    src_ref=out_ref.at[pl.ds(q_out_r * R, R), :],
                    dst_ref=out_ref.at[pl.ds(q_out_r * R, R), :],
                    send_sem=rsend.at[s],
                    recv_sem=rrecv.at[s],
                    device_id=(nxt_x, my_y, nxt_z),
                    device_id_type=pl.DeviceIdType.MESH,
                )
                recv_r = pltpu.make_async_remote_copy(
                    src_ref=out_ref.at[pl.ds(q_in_r * R, R), :],
                    dst_ref=out_ref.at[pl.ds(q_in_r * R, R), :],
                    send_sem=rsend.at[s],
                    recv_sem=rrecv.at[s],
                    device_id=(prv_x, my_y, prv_z),
                    device_id_type=pl.DeviceIdType.MESH,
                )
                send_r.start()
                ring_sends.append(send_r)
            q_out_l = (p + s) % N_P
            q_in_l = (p + 1 + s) % N_P
            send_l = pltpu.make_async_remote_copy(
                src_ref=out_ref.at[pl.ds(q_out_l * R, R), :],
                dst_ref=out_ref.at[pl.ds(q_out_l * R, R), :],
                send_sem=lsend.at[s],
                recv_sem=lrecv.at[s],
                device_id=(prv_x, my_y, prv_z),
                device_id_type=pl.DeviceIdType.MESH,
            )
            recv_l = pltpu.make_async_remote_copy(
                src_ref=out_ref.at[pl.ds(q_in_l * R, R), :],
                dst_ref=out_ref.at[pl.ds(q_in_l * R, R), :],
                send_sem=lsend.at[s],
                recv_sem=lrecv.at[s],
                device_id=(nxt_x, my_y, nxt_z),
                device_id_type=pl.DeviceIdType.MESH,
            )
            send_l.start()
            ring_sends.append(send_l)
            if s < 3:
                recv_r.wait_recv()
            recv_l.wait_recv()

        for rd in y_sends:
            rd.wait_send()
        for rd in ring_sends:
            rd.wait_send()

    return pl.pallas_call(
        body,
        out_shape=jax.ShapeDtypeStruct((M_PER, D), jnp.float32),
        in_specs=[
            pl.BlockSpec(memory_space=pltpu.VMEM),
            pl.BlockSpec(memory_space=pltpu.VMEM),
        ],
        out_specs=pl.BlockSpec(memory_space=pltpu.VMEM),
        scratch_shapes=[
            pltpu.VMEM((N_Y - 1, R, D), jnp.float32),
            pltpu.SemaphoreType.DMA((N_Y - 1,)),
            pltpu.SemaphoreType.DMA((N_Y - 1,)),
            pltpu.SemaphoreType.DMA((3,)),
            pltpu.SemaphoreType.DMA((3,)),
            pltpu.SemaphoreType.DMA((4,)),
            pltpu.SemaphoreType.DMA((4,)),
        ],
        compiler_params=pltpu.CompilerParams(collective_id=0),
    )(part, gamma2)


# device time: 56018 ns/iter; 2.7512x vs baseline; 1.1587x over previous
import jax
import jax.numpy as jnp
from jax import lax
from jax.experimental import pallas as pl
from jax.experimental.pallas import tpu as pltpu

N_Y = 4
N_P = 8
M_PER = 1024
R = M_PER // N_P
W = R // 2
D = 1024
EPS = 1e-6


def kernel(partial, gamma):
    part = partial.reshape(N_Y * M_PER, D)
    gamma2 = gamma.reshape(1, D)

    def body(part_ref, gamma_ref, out_ref,
             ybuf, ysend, yrecv, rsend, rrecv, lsend, lrecv):
        my_x = lax.axis_index("x")
        my_y = lax.axis_index("y")
        my_z = lax.axis_index("z")

        p = jnp.where(my_x == 0, my_z, 7 - my_z)

        def coords_of(pos):
            xx = pos // 4
            zz = jnp.where(xx == 0, pos % 4, 7 - pos)
            return xx, zz

        nxt_x, nxt_z = coords_of((p + 1) % N_P)
        prv_x, prv_z = coords_of((p - 1) % N_P)

        barrier_sem = pltpu.get_barrier_semaphore()
        for o in range(1, N_Y):
            pl.semaphore_signal(
                barrier_sem, inc=1,
                device_id=(my_x, (my_y + o) % N_Y, my_z),
                device_id_type=pl.DeviceIdType.MESH,
            )
        for dev in ((nxt_x, my_y, nxt_z), (prv_x, my_y, prv_z)):
            pl.semaphore_signal(
                barrier_sem, inc=1,
                device_id=dev, device_id_type=pl.DeviceIdType.MESH,
            )
        pl.semaphore_wait(barrier_sem, 5)

        y_sends = {}
        for w in (0, 1):
            for o in range(1, N_Y):
                r = (my_y + o) % N_Y
                rd = pltpu.make_async_remote_copy(
                    src_ref=part_ref.at[
                        pl.ds(r * M_PER + p * R + w * W, W), :],
                    dst_ref=ybuf.at[w, o - 1],
                    send_sem=ysend.at[w * 3 + o - 1],
                    recv_sem=yrecv.at[w * 3 + o - 1],
                    device_id=(my_x, r, my_z),
                    device_id_type=pl.DeviceIdType.MESH,
                )
                y_sends[(w, o)] = rd
            if w == 0:
                for o in range(1, N_Y):
                    y_sends[(0, o)].start()
                for o in range(1, N_Y):
                    y_sends[(0, o)].wait_send()
            else:
                for o in range(1, N_Y):
                    y_sends[(1, o)].start()

        def ring_step(w, s):
            sends, recvs = [], []
            base = p * R + 0
            if s < 3:
                q_out = (p - s) % N_P
                q_in = (p - 1 - s) % N_P
                sends.append(pltpu.make_async_remote_copy(
                    src_ref=out_ref.at[pl.ds(q_out * R + w * W, W), :],
                    dst_ref=out_ref.at[pl.ds(q_out * R + w * W, W), :],
                    send_sem=rsend.at[w * 3 + s],
                    recv_sem=rrecv.at[w * 3 + s],
                    device_id=(nxt_x, my_y, nxt_z),
                    device_id_type=pl.DeviceIdType.MESH,
                ))
                recvs.append(pltpu.make_async_remote_copy(
                    src_ref=out_ref.at[pl.ds(q_in * R + w * W, W), :],
                    dst_ref=out_ref.at[pl.ds(q_in * R + w * W, W), :],
                    send_sem=rsend.at[w * 3 + s],
                    recv_sem=rrecv.at[w * 3 + s],
                    device_id=(prv_x, my_y, prv_z),
                    device_id_type=pl.DeviceIdType.MESH,
                ))
            q_out = (p + s) % N_P
            q_in = (p + 1 + s) % N_P
            sends.append(pltpu.make_async_remote_copy(
                src_ref=out_ref.at[pl.ds(q_out * R + w * W, W), :],
                dst_ref=out_ref.at[pl.ds(q_out * R + w * W, W), :],
                send_sem=lsend.at[w * 4 + s],
                recv_sem=lrecv.at[w * 4 + s],
                device_id=(prv_x, my_y, prv_z),
                device_id_type=pl.DeviceIdType.MESH,
            ))
            recvs.append(pltpu.make_async_remote_copy(
                src_ref=out_ref.at[pl.ds(q_in * R + w * W, W), :],
                dst_ref=out_ref.at[pl.ds(q_in * R + w * W, W), :],
                send_sem=lsend.at[w * 4 + s],
                recv_sem=lrecv.at[w * 4 + s],
                device_id=(nxt_x, my_y, nxt_z),
                device_id_type=pl.DeviceIdType.MESH,
            ))
            return sends, recvs

        ring_sends = []
        ring_recvs = {}
        for w in (0, 1):
            for o in range(1, N_Y):
                y_sends[(w, o)].wait_recv()
            own = part_ref[pl.ds(my_y * M_PER + p * R + w * W, W), :]
            y_loc = own + ybuf[w, 0] + ybuf[w, 1] + ybuf[w, 2]
            ms = jnp.mean(y_loc * y_loc, axis=-1, keepdims=True)
            out_ref[pl.ds(p * R + w * W, W), :] = (
                y_loc * lax.rsqrt(ms + EPS) * gamma_ref[...]
            )
            sends, recvs = ring_step(w, 0)
            for rd in sends:
                rd.start()
            ring_sends += sends
            ring_recvs[(w, 0)] = recvs
        for s in range(1, 4):
            for w in (0, 1):
                for rd in ring_recvs[(w, s - 1)]:
                    rd.wait_recv()
                sends, recvs = ring_step(w, s)
                for rd in sends:
                    rd.start()
                ring_sends += sends
                ring_recvs[(w, s)] = recvs
        for w in (0, 1):
            for rd in ring_recvs[(w, 3)]:
                rd.wait_recv()

        for o in range(1, N_Y):
            y_sends[(1, o)].wait_send()
        for rd in ring_sends:
            rd.wait_send()

    return pl.pallas_call(
        body,
        out_shape=jax.ShapeDtypeStruct((M_PER, D), jnp.float32),
        in_specs=[
            pl.BlockSpec(memory_space=pltpu.VMEM),
            pl.BlockSpec(memory_space=pltpu.VMEM),
        ],
        out_specs=pl.BlockSpec(memory_space=pltpu.VMEM),
        scratch_shapes=[
            pltpu.VMEM((2, N_Y - 1, W, D), jnp.float32),
            pltpu.SemaphoreType.DMA((6,)),
            pltpu.SemaphoreType.DMA((6,)),
            pltpu.SemaphoreType.DMA((6,)),
            pltpu.SemaphoreType.DMA((6,)),
            pltpu.SemaphoreType.DMA((8,)),
            pltpu.SemaphoreType.DMA((8,)),
        ],
        compiler_params=pltpu.CompilerParams(collective_id=0),
    )(part, gamma2)
